# baseline (speedup 1.0000x reference)
import jax
import jax.numpy as jnp
from jax.experimental import pallas as pl
from jax.experimental.pallas import tpu as pltpu

_CANDIDATES = 1000
_PAD = 1024
_IOU_THRESHOLD = 0.45
_SCORE_THRESHOLD = 0.01
_WIDTH = 512.0
_HEIGHT = 512.0
_NFG = 80


def _class_max_kernel(fg_ref, smax_ref, lbl_ref):
    fg = fg_ref[...]
    m = jnp.max(fg, axis=1, keepdims=True)
    cid = jax.lax.broadcasted_iota(jnp.int32, fg.shape, 1)
    lbl = jnp.min(jnp.where(fg == m, cid, _NFG), axis=1, keepdims=True)
    smax_ref[...] = m
    lbl_ref[...] = lbl


def _nms_kernel(bt_ref, bc_ref, sc_ref, lbr_ref, lbc_ref,
                ob_ref, ol_ref, op_ref, iou_ref):
    bt = bt_ref[...]
    bc = bc_ref[...]
    sc = sc_ref[...]
    lbr = lbr_ref[...]
    lbc = lbc_ref[...]

    max_coord = jnp.max(bc)
    shift = max_coord + 1.0

    offs_r = lbr.astype(jnp.float32) * shift
    offs_c = lbc * shift

    x1r = bt[0:1, :] + offs_r
    y1r = bt[1:2, :] + offs_r
    x2r = bt[2:3, :] + offs_r
    y2r = bt[3:4, :] + offs_r
    x1c = bc[:, 0:1] + offs_c
    y1c = bc[:, 1:2] + offs_c
    x2c = bc[:, 2:3] + offs_c
    y2c = bc[:, 3:4] + offs_c

    w = jnp.maximum(jnp.minimum(x2c, x2r) - jnp.maximum(x1c, x1r), 0.0)
    h = jnp.maximum(jnp.minimum(y2c, y2r) - jnp.maximum(y1c, y1r), 0.0)
    inter = w * h
    area_r = (jnp.maximum(x2r - x1r, 0.0) * jnp.maximum(y2r - y1r, 0.0))
    area_c = (jnp.maximum(x2c - x1c, 0.0) * jnp.maximum(y2c - y1c, 0.0))
    union = area_c + area_r - inter
    del inter, union

    idx = jax.lax.broadcasted_iota(jnp.int32, (1, _PAD), 1)
    keepf = jnp.where(sc > _SCORE_THRESHOLD, 1.0, 0.0)

    blk_rows = 8
    row_iota = jax.lax.broadcasted_iota(jnp.int32, (blk_rows, 1), 0)

    def body(k, keepf):
        base = k * blk_rows
        blkf = jnp.where(
            iou_ref[pl.ds(base, blk_rows), :] > _IOU_THRESHOLD, 1.0, 0.0)
        rel = idx - base
        kblk = [jnp.max(jnp.where(rel == j, keepf, 0.0)).reshape(1, 1)
                for j in range(blk_rows)]
        onehot = jnp.where(rel == row_iota, 1.0, 0.0)
        s = jax.lax.dot_general(blkf, onehot, (((1,), (1,)), ((), ())),
                                preferred_element_type=jnp.float32)
        lk = []
        for j in range(blk_rows):
            acc = kblk[j]
            for m in range(j):
                acc = acc * (1.0 - lk[m] * s[j:j + 1, m:m + 1])
            lk.append(acc)
        lkcol = jnp.concatenate(lk, axis=0)
        sup = jnp.max(jnp.where(rel > row_iota, blkf * lkcol, 0.0),
                      axis=0, keepdims=True)
        return keepf * (1.0 - jnp.minimum(sup, 1.0))

    keepf = jax.lax.fori_loop(0, _CANDIDATES // blk_rows, body, keepf)

    coord = jax.lax.broadcasted_iota(jnp.int32, (4, 1), 0)
    scale = jnp.where(coord % 2 == 0, _WIDTH, _HEIGHT)
    ob_ref[...] = bt * keepf * scale
    ol_ref[...] = jnp.where(keepf > 0.0, lbr + 1, 0)
    op_ref[...] = sc * keepf


def kernel(scores, boxes):
    s = scores[0]
    b = boxes[0]
    n = s.shape[0]

    fg = s[:, 1:]
    fgp = jnp.pad(fg, ((0, 0), (0, 128 - _NFG)), constant_values=-1.0)

    smax, lbl = pl.pallas_call(
        _class_max_kernel,
        out_shape=(
            jax.ShapeDtypeStruct((n, 1), jnp.float32),
            jax.ShapeDtypeStruct((n, 1), jnp.int32),
        ),
    )(fgp)
    scores_single = smax[:, 0]
    labels = lbl[:, 0]

    scores_topk, topk_idx = jax.lax.top_k(scores_single, _CANDIDATES)
    boxes_topk = jnp.take(b, topk_idx, axis=0)
    labels_topk = jnp.take(labels, topk_idx, axis=0)

    pad = _PAD - _CANDIDATES
    sc_p = jnp.pad(scores_topk, (0, pad))
    bx_p = jnp.pad(boxes_topk, ((0, pad), (0, 0)))
    lb_p = jnp.pad(labels_topk, (0, pad))

    ob, ol, op = pl.pallas_call(
        _nms_kernel,
        out_shape=(
            jax.ShapeDtypeStruct((4, _PAD), jnp.float32),
            jax.ShapeDtypeStruct((1, _PAD), jnp.int32),
            jax.ShapeDtypeStruct((1, _PAD), jnp.float32),
        ),
        scratch_shapes=[pltpu.VMEM((_PAD, _PAD), jnp.float32)],
    )(bx_p.T, bx_p, sc_p[None, :], lb_p[None, :],
      lb_p.astype(jnp.float32)[:, None])

    selected_boxes = ob.T[:_CANDIDATES]
    selected_labels = ol[0, :_CANDIDATES]
    selected_probs = op[0, :_CANDIDATES]
    return selected_boxes, selected_labels, selected_probs


# speedup vs baseline: 1.3564x; 1.3564x over previous
import jax
import jax.numpy as jnp
from jax.experimental import pallas as pl
from jax.experimental.pallas import tpu as pltpu

_CANDIDATES = 1000
_PAD = 1024
_IOU_THRESHOLD = 0.45
_SCORE_THRESHOLD = 0.01
_WIDTH = 512.0
_HEIGHT = 512.0
_NFG = 80


def _class_max_kernel(st_ref, smax_ref, lbl_ref):
    st = st_ref[...]
    cid = jax.lax.broadcasted_iota(jnp.int32, st.shape, 0)
    fg = jnp.where(cid >= 1, st, -1.0)
    m = jnp.max(fg, axis=0, keepdims=True)
    lbl = jnp.min(jnp.where(fg == m, cid - 1, _NFG), axis=0, keepdims=True)
    smax_ref[...] = m
    lbl_ref[...] = lbl


def _nms_kernel(bt_ref, bc_ref, sc_ref, lbr_ref, lbc_ref,
                ob_ref, ol_ref, op_ref, iou_ref):
    bt = bt_ref[...]
    bc = bc_ref[...]
    sc = sc_ref[...]
    lbr = lbr_ref[...]
    lbc = lbc_ref[...]

    max_coord = jnp.max(bc)
    shift = max_coord + 1.0

    offs_r = lbr.astype(jnp.float32) * shift
    offs_c = lbc * shift

    x1r = bt[0:1, :] + offs_r
    y1r = bt[1:2, :] + offs_r
    x2r = bt[2:3, :] + offs_r
    y2r = bt[3:4, :] + offs_r
    x1c = bc[:, 0:1] + offs_c
    y1c = bc[:, 1:2] + offs_c
    x2c = bc[:, 2:3] + offs_c
    y2c = bc[:, 3:4] + offs_c

    w = jnp.maximum(jnp.minimum(x2c, x2r) - jnp.maximum(x1c, x1r), 0.0)
    h = jnp.maximum(jnp.minimum(y2c, y2r) - jnp.maximum(y1c, y1r), 0.0)
    inter = w * h
    area_r = (jnp.maximum(x2r - x1r, 0.0) * jnp.maximum(y2r - y1r, 0.0))
    area_c = (jnp.maximum(x2c - x1c, 0.0) * jnp.maximum(y2c - y1c, 0.0))
    union = area_c + area_r - inter
    iou_ref[...] = inter / jnp.maximum(union, 1e-9)

    idx = jax.lax.broadcasted_iota(jnp.int32, (1, _PAD), 1)
    keepf = jnp.where(sc > _SCORE_THRESHOLD, 1.0, 0.0)

    blk_rows = 8
    row_iota = jax.lax.broadcasted_iota(jnp.int32, (blk_rows, 1), 0)

    def body(k, keepf):
        base = k * blk_rows
        blkf = jnp.where(
            iou_ref[pl.ds(base, blk_rows), :] > _IOU_THRESHOLD, 1.0, 0.0)
        rel = idx - base
        kblk = [jnp.max(jnp.where(rel == j, keepf, 0.0)).reshape(1, 1)
                for j in range(blk_rows)]
        onehot = jnp.where(rel == row_iota, 1.0, 0.0)
        s = jax.lax.dot_general(blkf, onehot, (((1,), (1,)), ((), ())),
                                preferred_element_type=jnp.float32)
        lk = []
        for j in range(blk_rows):
            acc = kblk[j]
            for m in range(j):
                acc = acc * (1.0 - lk[m] * s[j:j + 1, m:m + 1])
            lk.append(acc)
        lkcol = jnp.concatenate(lk, axis=0)
        sup = jnp.max(jnp.where(rel > row_iota, blkf * lkcol, 0.0),
                      axis=0, keepdims=True)
        return keepf * (1.0 - jnp.minimum(sup, 1.0))

    keepf = jax.lax.fori_loop(0, _CANDIDATES // blk_rows, body, keepf)

    coord = jax.lax.broadcasted_iota(jnp.int32, (4, 1), 0)
    scale = jnp.where(coord % 2 == 0, _WIDTH, _HEIGHT)
    ob_ref[...] = bt * keepf * scale
    ol_ref[...] = jnp.where(keepf > 0.0, lbr + 1, 0)
    op_ref[...] = sc * keepf


def kernel(scores, boxes):
    s = scores[0]
    b = boxes[0]
    n = s.shape[0]

    smax, lbl = pl.pallas_call(
        _class_max_kernel,
        out_shape=(
            jax.ShapeDtypeStruct((1, n), jnp.float32),
            jax.ShapeDtypeStruct((1, n), jnp.int32),
        ),
    )(s.T)
    scores_single = smax[0]
    labels = lbl[0]

    scores_topk, topk_idx = jax.lax.top_k(scores_single, _CANDIDATES)
    boxes_topk = jnp.take(b, topk_idx, axis=0)
    labels_topk = jnp.take(labels, topk_idx, axis=0)

    pad = _PAD - _CANDIDATES
    sc_p = jnp.pad(scores_topk, (0, pad))
    bx_p = jnp.pad(boxes_topk, ((0, pad), (0, 0)))
    lb_p = jnp.pad(labels_topk, (0, pad))

    ob, ol, op = pl.pallas_call(
        _nms_kernel,
        out_shape=(
            jax.ShapeDtypeStruct((4, _PAD), jnp.float32),
            jax.ShapeDtypeStruct((1, _PAD), jnp.int32),
            jax.ShapeDtypeStruct((1, _PAD), jnp.float32),
        ),
        scratch_shapes=[pltpu.VMEM((_PAD, _PAD), jnp.float32)],
    )(bx_p.T, bx_p, sc_p[None, :], lb_p[None, :],
      lb_p.astype(jnp.float32)[:, None])

    selected_boxes = ob.T[:_CANDIDATES]
    selected_labels = ol[0, :_CANDIDATES]
    selected_probs = op[0, :_CANDIDATES]
    return selected_boxes, selected_labels, selected_probs
